# Initial kernel scaffold; baseline (speedup 1.0000x reference)
import jax
import jax.numpy as jnp
from jax.experimental import pallas as pl


def kernel(target, n_classes):
    raise NotImplementedError("write your pallas kernel here")


# speedup vs baseline: 57.5559x; 57.5559x over previous
import functools

import jax
import jax.numpy as jnp
from jax import lax
from jax.experimental import pallas as pl
from jax.experimental.pallas import tpu as pltpu
from jax.experimental.pallas import tpu_sc as plsc

N_BINS = 150
BIN_PAD = 152
NC, NS, L = 2, 16, 16
NW = NC * NS
CHUNK = 32768


def _sc_partial_hists(flat):
    n = flat.shape[0]
    per_w = n // NW
    n_chunks = per_w // CHUNK
    assert per_w % CHUNK == 0 and n % NW == 0

    mesh = plsc.VectorSubcoreMesh(core_axis_name="c", subcore_axis_name="s")

    @functools.partial(
        pl.kernel,
        out_type=jax.ShapeDtypeStruct((NW, L * BIN_PAD), jnp.float32),
        mesh=mesh,
        compiler_params=pltpu.CompilerParams(
            needs_layout_passes=False,
            use_tc_tiling_on_sc=False,
        ),
        scratch_types=[
            pltpu.VMEM((CHUNK,), jnp.float32),
            pltpu.VMEM((CHUNK,), jnp.float32),
            pltpu.VMEM((L * BIN_PAD,), jnp.float32),
            pltpu.SemaphoreType.DMA,
            pltpu.SemaphoreType.DMA,
        ],
    )
    def sc_hist(x_hbm, out_hbm, buf0, buf1, hist, sem0, sem1):
        wid = lax.axis_index("s") * NC + lax.axis_index("c")
        base = wid * per_w
        bufs = (buf0, buf1)
        sems = (sem0, sem1)

        zeros = jnp.zeros((L,), jnp.float32)

        def zero_body(i, carry):
            hist[pl.ds(i * L, L)] = zeros
            return carry

        lax.fori_loop(0, (L * BIN_PAD) // L, zero_body, 0)

        lane_off = lax.iota(jnp.int32, L) * BIN_PAD
        ones = jnp.ones((L,), jnp.float32)

        def copy(c):
            b = c % 2
            return pltpu.make_async_copy(
                x_hbm.at[pl.ds(base + c * CHUNK, CHUNK)], bufs[b], sems[b]
            )

        def process(buf):
            def body(j, carry):
                off = j * (8 * L)
                for k in range(8):
                    v = buf[pl.ds(off + k * L, L)]
                    idx = lane_off + v.astype(jnp.int32)
                    plsc.addupdate_scatter(hist, [idx], ones)
                return carry

            lax.fori_loop(0, CHUNK // (8 * L), body, 0)

        copy(0).start()
        for c in range(n_chunks):
            if c + 1 < n_chunks:
                copy(c + 1).start()
            copy(c).wait()
            process(bufs[c % 2])

        pltpu.sync_copy(hist, out_hbm.at[wid])

    return sc_hist(flat)


def _tc_finish(partials):
    def body(p_ref, o_ref):
        h = jnp.sum(p_ref[...], axis=0)
        total = jnp.sum(h)
        norm = h / total + 1.1
        w = 1.0 / jnp.log(norm)
        w = jnp.where(h == 0.0, 0.0, w)
        o_ref[...] = w.reshape(1, BIN_PAD)

    return pl.pallas_call(
        body,
        out_shape=jax.ShapeDtypeStruct((1, BIN_PAD), jnp.float32),
    )(partials)


def kernel(target, n_classes):
    flat = target.reshape(-1).astype(jnp.float32)
    partials = _sc_partial_hists(flat)
    weights = _tc_finish(partials.reshape(NW * L, BIN_PAD))
    return weights[0, :N_BINS]
